# baseline (device time: 84552 ns/iter reference)
import functools

import jax
import jax.numpy as jnp
from jax import lax
from jax.experimental import pallas as pl
from jax.experimental.pallas import tpu as pltpu

N_DEV = 4


def kernel(x, w_mat):
    m_per, k = x.shape
    _, n_per = w_mat.shape
    half_k = k // 2

    def body(x_ref, w_ref, qw_ref, s_ref, out_ref, qwg_ref, sc_ref,
             wdq_ref, obq_ref, obs_ref, obq_in_ref, obs_in_ref, send_sems,
             recv_sems):
        me = lax.axis_index("i")
        left = (me - 1) % N_DEV
        right = (me + 1) % N_DEV
        opp = (me + 2) % N_DEV

        barrier_sem = pltpu.get_barrier_semaphore()
        for nbr in (left, right, opp):
            pl.semaphore_signal(
                barrier_sem, inc=1,
                device_id=(nbr,), device_id_type=pl.DeviceIdType.MESH,
            )
        pl.semaphore_wait(barrier_sem, 3)

        def mk(src, dst, i, dev):
            return pltpu.make_async_remote_copy(
                src_ref=src, dst_ref=dst,
                send_sem=send_sems.at[i], recv_sem=recv_sems.at[i],
                device_id=(dev,), device_id_type=pl.DeviceIdType.MESH,
            )

        s9 = mk(s_ref, sc_ref.at[0], 9, right)
        s10 = mk(s_ref, sc_ref.at[1], 10, left)
        s11 = mk(s_ref, sc_ref.at[2], 11, opp)
        s9.start()
        s10.start()
        s11.start()
        qw_h0 = qw_ref.at[pl.ds(0, half_k)]
        qw_h1 = qw_ref.at[pl.ds(half_k, half_k)]
        s0 = mk(qw_h0, qwg_ref.at[0, 0], 0, right)
        s1 = mk(qw_h1, qwg_ref.at[0, 1], 1, right)
        s2 = mk(qw_h1, qwg_ref.at[1, 1], 2, left)
        s3 = mk(qw_h0, qwg_ref.at[1, 0], 3, left)
        s0.start()
        s2.start()
        s1.start()
        s3.start()

        def dots(w0, w1):
            acc = jnp.dot(x_ref[:, pl.ds(0, half_k)], w0,
                          preferred_element_type=jnp.float32)
            acc += jnp.dot(x_ref[:, pl.ds(half_k, half_k)], w1,
                           preferred_element_type=jnp.float32)
            return acc

        def qblock(slot, sc_slot):
            acc = dots(wdq_ref[slot, 0], wdq_ref[slot, 1])
            scale = sc_ref[sc_slot, 0, :] * (1.0 / 127.0)
            return jnp.maximum(acc * scale[None, :], 0.0)

        def qstore(i, acc):
            sb = jnp.maximum(jnp.max(acc, axis=0), 1e-30)
            obs_ref[i, 0, :] = sb
            obq_ref[i, :, :] = jnp.round(
                acc * (127.0 / sb)[None, :]
            ).astype(jnp.int8)

        def dq_store(i, row_start):
            scale = obs_in_ref[i, 0, :] * (1.0 / 127.0)
            out_ref[pl.ds(row_start, m_per), :] = (
                obq_in_ref[i].astype(jnp.float32) * scale[None, :]
            ).astype(jnp.bfloat16)

        acc = dots(w_ref[pl.ds(0, half_k), :], w_ref[pl.ds(half_k, half_k), :])
        out_ref[pl.ds(me * m_per, m_per), :] = jnp.maximum(
            acc, 0.0
        ).astype(jnp.bfloat16)

        s0.wait_recv()
        s4 = mk(qwg_ref.at[0, 0], qwg_ref.at[2, 0], 4, right)
        s4.start()
        s2.wait_recv()
        s5 = mk(qwg_ref.at[1, 1], qwg_ref.at[2, 1], 5, left)
        s5.start()

        wdq_ref[0, 0, :, :] = qwg_ref[0, 0].astype(jnp.bfloat16)
        wdq_ref[1, 1, :, :] = qwg_ref[1, 1].astype(jnp.bfloat16)

        s3.wait_recv()
        wdq_ref[1, 0, :, :] = qwg_ref[1, 0].astype(jnp.bfloat16)
        s10.wait_recv()
        qstore(0, qblock(1, 1))
        s6 = mk(obq_ref.at[0], obq_in_ref.at[0], 6, right)
        s6.start()
        s12 = mk(obs_ref.at[0], obs_in_ref.at[0], 12, right)
        s12.start()

        s1.wait_recv()
        wdq_ref[0, 1, :, :] = qwg_ref[0, 1].astype(jnp.bfloat16)
        s9.wait_recv()
        qstore(1, qblock(0, 0))
        s7 = mk(obq_ref.at[1], obq_in_ref.at[1], 7, left)
        s7.start()
        s13 = mk(obs_ref.at[1], obs_in_ref.at[1], 13, left)
        s13.start()

        s4.wait_recv()
        wdq_ref[2, 0, :, :] = qwg_ref[2, 0].astype(jnp.bfloat16)
        s5.wait_recv()
        wdq_ref[2, 1, :, :] = qwg_ref[2, 1].astype(jnp.bfloat16)
        s11.wait_recv()
        qstore(2, qblock(2, 2))
        s8 = mk(obq_ref.at[2], obq_in_ref.at[2], 8, opp)
        s8.start()
        s14 = mk(obs_ref.at[2], obs_in_ref.at[2], 14, opp)
        s14.start()

        s6.wait_recv()
        s12.wait_recv()
        dq_store(0, left * m_per)
        s7.wait_recv()
        s13.wait_recv()
        dq_store(1, right * m_per)
        s8.wait_recv()
        s14.wait_recv()
        dq_store(2, opp * m_per)

        for s in (s0, s1, s2, s3, s4, s5, s6, s7, s8, s9, s10, s11, s12,
                  s13, s14):
            s.wait_send()

        @functools.partial(
            pl.run_scoped, second_barrier=pltpu.SemaphoreType.REGULAR
        )
        def _(second_barrier):
            for nbr in (left, right):
                pl.semaphore_signal(
                    second_barrier, inc=1,
                    device_id=(nbr,), device_id_type=pl.DeviceIdType.MESH,
                )
            pl.semaphore_wait(second_barrier, 2)

    wb = w_mat.astype(jnp.bfloat16)
    s = jnp.maximum(jnp.max(jnp.abs(w_mat), axis=0), 1e-30)
    qw = jnp.round(w_mat * (127.0 / s)).astype(jnp.int8)
    s_tile = jnp.broadcast_to(s.astype(jnp.float32), (8, n_per))

    return pl.pallas_call(
        body,
        out_shape=jax.ShapeDtypeStruct((N_DEV * m_per, n_per), jnp.bfloat16),
        in_specs=[
            pl.BlockSpec(memory_space=pltpu.VMEM),
            pl.BlockSpec(memory_space=pltpu.VMEM),
            pl.BlockSpec(memory_space=pltpu.VMEM),
            pl.BlockSpec(memory_space=pltpu.VMEM),
        ],
        out_specs=pl.BlockSpec(memory_space=pltpu.VMEM),
        scratch_shapes=[
            pltpu.VMEM((3, 2, half_k, n_per), jnp.int8),
            pltpu.VMEM((3, 8, n_per), jnp.float32),
            pltpu.VMEM((3, 2, half_k, n_per), jnp.bfloat16),
            pltpu.VMEM((3, m_per, n_per), jnp.int8),
            pltpu.VMEM((3, 8, n_per), jnp.float32),
            pltpu.VMEM((3, m_per, n_per), jnp.int8),
            pltpu.VMEM((3, 8, n_per), jnp.float32),
            pltpu.SemaphoreType.DMA((15,)),
            pltpu.SemaphoreType.DMA((15,)),
        ],
        compiler_params=pltpu.CompilerParams(
            collective_id=0, vmem_limit_bytes=60 * 1024 * 1024
        ),
    )(x.astype(jnp.bfloat16), wb, qw, s_tile)


# device time: 76082 ns/iter; 1.1113x vs baseline; 1.1113x over previous
import functools

import jax
import jax.numpy as jnp
from jax import lax
from jax.experimental import pallas as pl
from jax.experimental.pallas import tpu as pltpu

N_DEV = 4


def kernel(x, w_mat):
    m_per, k = x.shape
    _, n_per = w_mat.shape
    half_k = k // 2

    def body(x_ref, w_ref, qw_ref, s_ref, out_ref, xb_ref, qwg_ref, sc_ref,
             wdq_ref, obq_ref, obs_ref, obq_in_ref, obs_in_ref, send_sems,
             recv_sems):
        me = lax.axis_index("i")
        left = (me - 1) % N_DEV
        right = (me + 1) % N_DEV
        opp = (me + 2) % N_DEV

        barrier_sem = pltpu.get_barrier_semaphore()
        for nbr in (left, right, opp):
            pl.semaphore_signal(
                barrier_sem, inc=1,
                device_id=(nbr,), device_id_type=pl.DeviceIdType.MESH,
            )
        pl.semaphore_wait(barrier_sem, 3)

        def mk(src, dst, i, dev):
            return pltpu.make_async_remote_copy(
                src_ref=src, dst_ref=dst,
                send_sem=send_sems.at[i], recv_sem=recv_sems.at[i],
                device_id=(dev,), device_id_type=pl.DeviceIdType.MESH,
            )

        s9 = mk(s_ref, sc_ref.at[0], 9, right)
        s10 = mk(s_ref, sc_ref.at[1], 10, left)
        s11 = mk(s_ref, sc_ref.at[2], 11, opp)
        s9.start()
        s10.start()
        s11.start()
        qw_h0 = qw_ref.at[pl.ds(0, half_k)]
        qw_h1 = qw_ref.at[pl.ds(half_k, half_k)]
        s0 = mk(qw_h0, qwg_ref.at[0, 0], 0, right)
        s1 = mk(qw_h1, qwg_ref.at[0, 1], 1, right)
        s2 = mk(qw_h1, qwg_ref.at[1, 1], 2, left)
        s3 = mk(qw_h0, qwg_ref.at[1, 0], 3, left)
        s0.start()
        s2.start()
        s1.start()
        s3.start()

        xb_ref[:, :] = x_ref[:, :].astype(jnp.bfloat16)

        def dots(w0, w1):
            acc = jnp.dot(xb_ref[:, pl.ds(0, half_k)], w0,
                          preferred_element_type=jnp.float32)
            acc += jnp.dot(xb_ref[:, pl.ds(half_k, half_k)], w1,
                           preferred_element_type=jnp.float32)
            return acc

        def qblock(slot, sc_slot):
            acc = dots(wdq_ref[slot, 0], wdq_ref[slot, 1])
            scale = sc_ref[sc_slot, 0, :] * (1.0 / 127.0)
            return jnp.maximum(acc * scale[None, :], 0.0)

        def qstore(i, acc):
            sb = jnp.maximum(jnp.max(acc, axis=0), 1e-30)
            obs_ref[i, 0, :] = sb
            obq_ref[i, :, :] = jnp.round(
                acc * (127.0 / sb)[None, :]
            ).astype(jnp.int8)

        def dq_store(i, row_start):
            scale = obs_in_ref[i, 0, :] * (1.0 / 127.0)
            out_ref[pl.ds(row_start, m_per), :] = (
                obq_in_ref[i].astype(jnp.float32) * scale[None, :]
            ).astype(jnp.bfloat16)

        acc = dots(w_ref[pl.ds(0, half_k), :], w_ref[pl.ds(half_k, half_k), :])
        out_ref[pl.ds(me * m_per, m_per), :] = jnp.maximum(
            acc, 0.0
        ).astype(jnp.bfloat16)

        s0.wait_recv()
        s4 = mk(qwg_ref.at[0, 0], qwg_ref.at[2, 0], 4, right)
        s4.start()
        s2.wait_recv()
        s5 = mk(qwg_ref.at[1, 1], qwg_ref.at[2, 1], 5, left)
        s5.start()

        wdq_ref[0, 0, :, :] = qwg_ref[0, 0].astype(jnp.bfloat16)
        wdq_ref[1, 1, :, :] = qwg_ref[1, 1].astype(jnp.bfloat16)

        s3.wait_recv()
        wdq_ref[1, 0, :, :] = qwg_ref[1, 0].astype(jnp.bfloat16)
        s10.wait_recv()
        qstore(0, qblock(1, 1))
        s6 = mk(obq_ref.at[0], obq_in_ref.at[0], 6, right)
        s6.start()
        s12 = mk(obs_ref.at[0], obs_in_ref.at[0], 12, right)
        s12.start()

        s1.wait_recv()
        wdq_ref[0, 1, :, :] = qwg_ref[0, 1].astype(jnp.bfloat16)
        s9.wait_recv()
        qstore(1, qblock(0, 0))
        s7 = mk(obq_ref.at[1], obq_in_ref.at[1], 7, left)
        s7.start()
        s13 = mk(obs_ref.at[1], obs_in_ref.at[1], 13, left)
        s13.start()

        s4.wait_recv()
        wdq_ref[2, 0, :, :] = qwg_ref[2, 0].astype(jnp.bfloat16)
        s5.wait_recv()
        wdq_ref[2, 1, :, :] = qwg_ref[2, 1].astype(jnp.bfloat16)
        s11.wait_recv()
        qstore(2, qblock(2, 2))
        s8 = mk(obq_ref.at[2], obq_in_ref.at[2], 8, opp)
        s8.start()
        s14 = mk(obs_ref.at[2], obs_in_ref.at[2], 14, opp)
        s14.start()

        s6.wait_recv()
        s12.wait_recv()
        dq_store(0, left * m_per)
        s7.wait_recv()
        s13.wait_recv()
        dq_store(1, right * m_per)
        s8.wait_recv()
        s14.wait_recv()
        dq_store(2, opp * m_per)

        for s in (s0, s1, s2, s3, s4, s5, s6, s7, s8, s9, s10, s11, s12,
                  s13, s14):
            s.wait_send()

        @functools.partial(
            pl.run_scoped, second_barrier=pltpu.SemaphoreType.REGULAR
        )
        def _(second_barrier):
            for nbr in (left, right):
                pl.semaphore_signal(
                    second_barrier, inc=1,
                    device_id=(nbr,), device_id_type=pl.DeviceIdType.MESH,
                )
            pl.semaphore_wait(second_barrier, 2)

    wb = w_mat.astype(jnp.bfloat16)
    s = jnp.maximum(jnp.max(jnp.abs(wb), axis=0).astype(jnp.float32), 1e-30)
    qw = jnp.round(wb.astype(jnp.float32) * (127.0 / s)).astype(jnp.int8)
    s_tile = jnp.broadcast_to(s, (8, n_per))

    return pl.pallas_call(
        body,
        out_shape=jax.ShapeDtypeStruct((N_DEV * m_per, n_per), jnp.bfloat16),
        in_specs=[
            pl.BlockSpec(memory_space=pltpu.VMEM),
            pl.BlockSpec(memory_space=pltpu.VMEM),
            pl.BlockSpec(memory_space=pltpu.VMEM),
            pl.BlockSpec(memory_space=pltpu.VMEM),
        ],
        out_specs=pl.BlockSpec(memory_space=pltpu.VMEM),
        scratch_shapes=[
            pltpu.VMEM((m_per, k), jnp.bfloat16),
            pltpu.VMEM((3, 2, half_k, n_per), jnp.int8),
            pltpu.VMEM((3, 8, n_per), jnp.float32),
            pltpu.VMEM((3, 2, half_k, n_per), jnp.bfloat16),
            pltpu.VMEM((3, m_per, n_per), jnp.int8),
            pltpu.VMEM((3, 8, n_per), jnp.float32),
            pltpu.VMEM((3, m_per, n_per), jnp.int8),
            pltpu.VMEM((3, 8, n_per), jnp.float32),
            pltpu.SemaphoreType.DMA((15,)),
            pltpu.SemaphoreType.DMA((15,)),
        ],
        compiler_params=pltpu.CompilerParams(
            collective_id=0, vmem_limit_bytes=60 * 1024 * 1024
        ),
    )(x, wb, qw, s_tile)


# device time: 75907 ns/iter; 1.1139x vs baseline; 1.0023x over previous
import functools

import jax
import jax.numpy as jnp
from jax import lax
from jax.experimental import pallas as pl
from jax.experimental.pallas import tpu as pltpu

N_DEV = 4


def kernel(x, w_mat):
    m_per, k = x.shape
    _, n_per = w_mat.shape
    half_k = k // 2

    def body(x_ref, w_ref, qw_ref, s_ref, out_ref, xb_ref, qwg_ref, sc_ref,
             wdq_ref, obq_ref, obs_ref, obq_in_ref, obs_in_ref, send_sems,
             recv_sems):
        me = lax.axis_index("i")
        left = (me - 1) % N_DEV
        right = (me + 1) % N_DEV
        opp = (me + 2) % N_DEV

        barrier_sem = pltpu.get_barrier_semaphore()
        for nbr in (left, right, opp):
            pl.semaphore_signal(
                barrier_sem, inc=1,
                device_id=(nbr,), device_id_type=pl.DeviceIdType.MESH,
            )
        pl.semaphore_wait(barrier_sem, 3)

        def mk(src, dst, i, dev):
            return pltpu.make_async_remote_copy(
                src_ref=src, dst_ref=dst,
                send_sem=send_sems.at[i], recv_sem=recv_sems.at[i],
                device_id=(dev,), device_id_type=pl.DeviceIdType.MESH,
            )

        s9 = mk(s_ref, sc_ref.at[0], 9, right)
        s10 = mk(s_ref, sc_ref.at[1], 10, left)
        s11 = mk(s_ref, sc_ref.at[2], 11, opp)
        s9.start()
        s10.start()
        s11.start()
        qw_h0 = qw_ref.at[pl.ds(0, half_k)]
        qw_h1 = qw_ref.at[pl.ds(half_k, half_k)]
        s0 = mk(qw_h0, qwg_ref.at[0, 0], 0, right)
        s1 = mk(qw_h1, qwg_ref.at[0, 1], 1, right)
        s2 = mk(qw_h1, qwg_ref.at[1, 1], 2, left)
        s3 = mk(qw_h0, qwg_ref.at[1, 0], 3, left)
        s0.start()
        s2.start()
        s1.start()
        s3.start()

        xb_ref[:, :] = x_ref[:, :].astype(jnp.bfloat16)

        def dots(w0, w1):
            acc = jnp.dot(xb_ref[:, pl.ds(0, half_k)], w0,
                          preferred_element_type=jnp.float32)
            acc += jnp.dot(xb_ref[:, pl.ds(half_k, half_k)], w1,
                           preferred_element_type=jnp.float32)
            return acc

        def qblock(slot, sc_slot):
            acc = dots(wdq_ref[slot, 0], wdq_ref[slot, 1])
            scale = sc_ref[sc_slot, 0, :] * (1.0 / 127.0)
            return jnp.maximum(acc * scale[None, :], 0.0)

        def qstore(i, acc):
            sb = jnp.maximum(jnp.max(acc, axis=0), 1e-30)
            obs_ref[i, 0, :] = sb
            obq_ref[i, :, :] = jnp.round(
                acc * (127.0 / sb)[None, :]
            ).astype(jnp.int8)

        def dq_store(i, row_start):
            scale = obs_in_ref[i, 0, :] * (1.0 / 127.0)
            out_ref[pl.ds(row_start, m_per), :] = (
                obq_in_ref[i].astype(jnp.float32) * scale[None, :]
            ).astype(jnp.bfloat16)

        acc = dots(w_ref[pl.ds(0, half_k), :], w_ref[pl.ds(half_k, half_k), :])
        out_ref[pl.ds(me * m_per, m_per), :] = jnp.maximum(
            acc, 0.0
        ).astype(jnp.bfloat16)

        s0.wait_recv()
        s4 = mk(qwg_ref.at[0, 0], qwg_ref.at[2, 0], 4, right)
        s4.start()
        s2.wait_recv()
        s5 = mk(qwg_ref.at[1, 1], qwg_ref.at[2, 1], 5, left)
        s5.start()

        wdq_ref[0, 0, :, :] = qwg_ref[0, 0].astype(jnp.bfloat16)
        wdq_ref[1, 1, :, :] = qwg_ref[1, 1].astype(jnp.bfloat16)

        s3.wait_recv()
        wdq_ref[1, 0, :, :] = qwg_ref[1, 0].astype(jnp.bfloat16)
        s10.wait_recv()
        qstore(0, qblock(1, 1))
        s6 = mk(obq_ref.at[0], obq_in_ref.at[0], 6, right)
        s6.start()
        s12 = mk(obs_ref.at[0], obs_in_ref.at[0], 12, right)
        s12.start()

        s1.wait_recv()
        wdq_ref[0, 1, :, :] = qwg_ref[0, 1].astype(jnp.bfloat16)
        s9.wait_recv()
        qstore(1, qblock(0, 0))
        s7 = mk(obq_ref.at[1], obq_in_ref.at[1], 7, left)
        s7.start()
        s13 = mk(obs_ref.at[1], obs_in_ref.at[1], 13, left)
        s13.start()

        half_m = m_per // 2
        s4.wait_recv()
        wdq_ref[2, 0, :, :] = qwg_ref[2, 0].astype(jnp.bfloat16)
        s5.wait_recv()
        wdq_ref[2, 1, :, :] = qwg_ref[2, 1].astype(jnp.bfloat16)
        s11.wait_recv()
        scale2 = sc_ref[2, 0, :] * (1.0 / 127.0)

        def half_rows(r):
            lo = r * half_m
            acc = jnp.dot(xb_ref[pl.ds(lo, half_m), pl.ds(0, half_k)],
                          wdq_ref[2, 0], preferred_element_type=jnp.float32)
            acc += jnp.dot(xb_ref[pl.ds(lo, half_m), pl.ds(half_k, half_k)],
                           wdq_ref[2, 1], preferred_element_type=jnp.float32)
            acc = jnp.maximum(acc * scale2[None, :], 0.0)
            sb = jnp.maximum(jnp.max(acc, axis=0), 1e-30)
            obs_ref[2 + r, 0, :] = sb
            obq_ref[2, pl.ds(lo, half_m), :] = jnp.round(
                acc * (127.0 / sb)[None, :]
            ).astype(jnp.int8)

        half_rows(0)
        s8a = mk(obq_ref.at[2, pl.ds(0, half_m)],
                 obq_in_ref.at[2, pl.ds(0, half_m)], 8, opp)
        s8a.start()
        s14a = mk(obs_ref.at[2], obs_in_ref.at[2], 14, opp)
        s14a.start()
        half_rows(1)
        s8b = mk(obq_ref.at[2, pl.ds(half_m, half_m)],
                 obq_in_ref.at[2, pl.ds(half_m, half_m)], 15, opp)
        s8b.start()
        s14b = mk(obs_ref.at[3], obs_in_ref.at[3], 16, opp)
        s14b.start()

        s6.wait_recv()
        s12.wait_recv()
        dq_store(0, left * m_per)
        s7.wait_recv()
        s13.wait_recv()
        dq_store(1, right * m_per)

        def dq_half(r):
            scale = obs_in_ref[2 + r, 0, :] * (1.0 / 127.0)
            lo = r * half_m
            out_ref[pl.ds(opp * m_per + lo, half_m), :] = (
                obq_in_ref[2, pl.ds(lo, half_m), :].astype(jnp.float32)
                * scale[None, :]
            ).astype(jnp.bfloat16)

        s8a.wait_recv()
        s14a.wait_recv()
        dq_half(0)
        s8b.wait_recv()
        s14b.wait_recv()
        dq_half(1)

        for s in (s0, s1, s2, s3, s4, s5, s6, s7, s8a, s8b, s9, s10, s11,
                  s12, s13, s14a, s14b):
            s.wait_send()

        @functools.partial(
            pl.run_scoped, second_barrier=pltpu.SemaphoreType.REGULAR
        )
        def _(second_barrier):
            for nbr in (left, right):
                pl.semaphore_signal(
                    second_barrier, inc=1,
                    device_id=(nbr,), device_id_type=pl.DeviceIdType.MESH,
                )
            pl.semaphore_wait(second_barrier, 2)

    wb = w_mat.astype(jnp.bfloat16)
    s = jnp.maximum(jnp.max(jnp.abs(wb), axis=0).astype(jnp.float32), 1e-30)
    qw = jnp.round(wb.astype(jnp.float32) * (127.0 / s)).astype(jnp.int8)
    s_tile = jnp.broadcast_to(s, (8, n_per))

    return pl.pallas_call(
        body,
        out_shape=jax.ShapeDtypeStruct((N_DEV * m_per, n_per), jnp.bfloat16),
        in_specs=[
            pl.BlockSpec(memory_space=pltpu.VMEM),
            pl.BlockSpec(memory_space=pltpu.VMEM),
            pl.BlockSpec(memory_space=pltpu.VMEM),
            pl.BlockSpec(memory_space=pltpu.VMEM),
        ],
        out_specs=pl.BlockSpec(memory_space=pltpu.VMEM),
        scratch_shapes=[
            pltpu.VMEM((m_per, k), jnp.bfloat16),
            pltpu.VMEM((3, 2, half_k, n_per), jnp.int8),
            pltpu.VMEM((3, 8, n_per), jnp.float32),
            pltpu.VMEM((3, 2, half_k, n_per), jnp.bfloat16),
            pltpu.VMEM((3, m_per, n_per), jnp.int8),
            pltpu.VMEM((4, 8, n_per), jnp.float32),
            pltpu.VMEM((3, m_per, n_per), jnp.int8),
            pltpu.VMEM((4, 8, n_per), jnp.float32),
            pltpu.SemaphoreType.DMA((17,)),
            pltpu.SemaphoreType.DMA((17,)),
        ],
        compiler_params=pltpu.CompilerParams(
            collective_id=0, vmem_limit_bytes=60 * 1024 * 1024
        ),
    )(x, wb, qw, s_tile)


# device time: 75728 ns/iter; 1.1165x vs baseline; 1.0024x over previous
import functools

import jax
import jax.numpy as jnp
from jax import lax
from jax.experimental import pallas as pl
from jax.experimental.pallas import tpu as pltpu

N_DEV = 4


def kernel(x, w_mat):
    m_per, k = x.shape
    _, n_per = w_mat.shape
    half_k = k // 2
    half_m = m_per // 2

    def body(x_ref, w_ref, qw_ref, s_ref, out_ref, xb_ref, qwg_ref, sc_ref,
             wdq_ref, obq_ref, obs_ref, obq_in_ref, obs_in_ref, send_sems,
             recv_sems):
        me = lax.axis_index("i")
        left = (me - 1) % N_DEV
        right = (me + 1) % N_DEV
        opp = (me + 2) % N_DEV

        barrier_sem = pltpu.get_barrier_semaphore()
        for nbr in (left, right, opp):
            pl.semaphore_signal(
                barrier_sem, inc=1,
                device_id=(nbr,), device_id_type=pl.DeviceIdType.MESH,
            )
        pl.semaphore_wait(barrier_sem, 3)

        def mk(src, dst, i, dev):
            return pltpu.make_async_remote_copy(
                src_ref=src, dst_ref=dst,
                send_sem=send_sems.at[i], recv_sem=recv_sems.at[i],
                device_id=(dev,), device_id_type=pl.DeviceIdType.MESH,
            )

        qw_h0 = qw_ref.at[pl.ds(0, half_k)]
        qw_h1 = qw_ref.at[pl.ds(half_k, half_k)]
        s0 = mk(qw_h0, qwg_ref.at[0, 0], 0, right)
        s1 = mk(qw_h1, qwg_ref.at[0, 1], 1, right)
        s2 = mk(qw_h1, qwg_ref.at[1, 1], 2, left)
        s3 = mk(qw_h0, qwg_ref.at[1, 0], 3, left)
        s0.start()
        s2.start()
        s1.start()
        s3.start()
        s9 = mk(s_ref, sc_ref.at[0], 9, right)
        s10 = mk(s_ref, sc_ref.at[1], 10, left)
        s11 = mk(s_ref, sc_ref.at[2], 11, opp)
        s9.start()
        s10.start()
        s11.start()

        xb_ref[:, :] = x_ref[:, :].astype(jnp.bfloat16)

        def dots(w0, w1):
            acc = jnp.dot(xb_ref[:, pl.ds(0, half_k)], w0,
                          preferred_element_type=jnp.float32)
            acc += jnp.dot(xb_ref[:, pl.ds(half_k, half_k)], w1,
                           preferred_element_type=jnp.float32)
            return acc

        def ob_half(i, slot, r):
            lo = r * half_m
            wscale = sc_ref[slot, 0, :] * (1.0 / 127.0)
            acc = jnp.dot(xb_ref[pl.ds(lo, half_m), pl.ds(0, half_k)],
                          wdq_ref[slot, 0],
                          preferred_element_type=jnp.float32)
            acc += jnp.dot(xb_ref[pl.ds(lo, half_m), pl.ds(half_k, half_k)],
                           wdq_ref[slot, 1],
                           preferred_element_type=jnp.float32)
            acc = jnp.maximum(acc * wscale[None, :], 0.0)
            sb = jnp.maximum(jnp.max(acc, axis=0), 1e-30)
            obs_ref[i, r, 0, :] = sb
            obq_ref[i, pl.ds(lo, half_m), :] = jnp.round(
                acc * (127.0 / sb)[None, :]
            ).astype(jnp.int8)

        def send_ob(i, r, data_idx, scale_idx, dev):
            lo = r * half_m
            d = mk(obq_ref.at[i, pl.ds(lo, half_m)],
                   obq_in_ref.at[i, pl.ds(lo, half_m)], data_idx, dev)
            d.start()
            sc = mk(obs_ref.at[i, r], obs_in_ref.at[i, r], scale_idx, dev)
            sc.start()
            return d, sc

        def dq_half(i, row_start, r):
            scale = obs_in_ref[i, r, 0, :] * (1.0 / 127.0)
            lo = r * half_m
            out_ref[pl.ds(row_start + lo, half_m), :] = (
                obq_in_ref[i, pl.ds(lo, half_m), :].astype(jnp.float32)
                * scale[None, :]
            ).astype(jnp.bfloat16)

        acc = dots(w_ref[pl.ds(0, half_k), :], w_ref[pl.ds(half_k, half_k), :])
        out_ref[pl.ds(me * m_per, m_per), :] = jnp.maximum(
            acc, 0.0
        ).astype(jnp.bfloat16)

        s0.wait_recv()
        s4 = mk(qwg_ref.at[0, 0], qwg_ref.at[2, 0], 4, right)
        s4.start()
        s2.wait_recv()
        s5 = mk(qwg_ref.at[1, 1], qwg_ref.at[2, 1], 5, left)
        s5.start()

        wdq_ref[0, 0, :, :] = qwg_ref[0, 0].astype(jnp.bfloat16)
        wdq_ref[1, 1, :, :] = qwg_ref[1, 1].astype(jnp.bfloat16)

        s3.wait_recv()
        wdq_ref[1, 0, :, :] = qwg_ref[1, 0].astype(jnp.bfloat16)
        s10.wait_recv()
        ob_half(0, 1, 0)
        s6a, s12a = send_ob(0, 0, 6, 15, right)
        ob_half(0, 1, 1)
        s6b, s12b = send_ob(0, 1, 12, 16, right)

        s1.wait_recv()
        wdq_ref[0, 1, :, :] = qwg_ref[0, 1].astype(jnp.bfloat16)
        s9.wait_recv()
        ob_half(1, 0, 0)
        s7a, s13a = send_ob(1, 0, 7, 17, left)
        ob_half(1, 0, 1)
        s7b, s13b = send_ob(1, 1, 13, 18, left)

        s4.wait_recv()
        wdq_ref[2, 0, :, :] = qwg_ref[2, 0].astype(jnp.bfloat16)
        s5.wait_recv()
        wdq_ref[2, 1, :, :] = qwg_ref[2, 1].astype(jnp.bfloat16)
        s11.wait_recv()
        ob_half(2, 2, 0)
        s8a, s14a = send_ob(2, 0, 8, 19, opp)
        ob_half(2, 2, 1)
        s8b, s14b = send_ob(2, 1, 14, 20, opp)

        s6a.wait_recv()
        s12a.wait_recv()
        dq_half(0, left * m_per, 0)
        s6b.wait_recv()
        s12b.wait_recv()
        dq_half(0, left * m_per, 1)
        s7a.wait_recv()
        s13a.wait_recv()
        dq_half(1, right * m_per, 0)
        s7b.wait_recv()
        s13b.wait_recv()
        dq_half(1, right * m_per, 1)
        s8a.wait_recv()
        s14a.wait_recv()
        dq_half(2, opp * m_per, 0)
        s8b.wait_recv()
        s14b.wait_recv()
        dq_half(2, opp * m_per, 1)

        for s in (s0, s1, s2, s3, s4, s5, s9, s10, s11, s6a, s6b, s7a, s7b,
                  s8a, s8b, s12a, s12b, s13a, s13b, s14a, s14b):
            s.wait_send()

        @functools.partial(
            pl.run_scoped, second_barrier=pltpu.SemaphoreType.REGULAR
        )
        def _(second_barrier):
            for nbr in (left, right):
                pl.semaphore_signal(
                    second_barrier, inc=1,
                    device_id=(nbr,), device_id_type=pl.DeviceIdType.MESH,
                )
            pl.semaphore_wait(second_barrier, 2)

    wb = w_mat.astype(jnp.bfloat16)
    s = jnp.maximum(jnp.max(jnp.abs(wb), axis=0).astype(jnp.float32), 1e-30)
    qw = jnp.round(wb.astype(jnp.float32) * (127.0 / s)).astype(jnp.int8)
    s_tile = jnp.broadcast_to(s, (8, n_per))

    return pl.pallas_call(
        body,
        out_shape=jax.ShapeDtypeStruct((N_DEV * m_per, n_per), jnp.bfloat16),
        in_specs=[
            pl.BlockSpec(memory_space=pltpu.VMEM),
            pl.BlockSpec(memory_space=pltpu.VMEM),
            pl.BlockSpec(memory_space=pltpu.VMEM),
            pl.BlockSpec(memory_space=pltpu.VMEM),
        ],
        out_specs=pl.BlockSpec(memory_space=pltpu.VMEM),
        scratch_shapes=[
            pltpu.VMEM((m_per, k), jnp.bfloat16),
            pltpu.VMEM((3, 2, half_k, n_per), jnp.int8),
            pltpu.VMEM((3, 8, n_per), jnp.float32),
            pltpu.VMEM((3, 2, half_k, n_per), jnp.bfloat16),
            pltpu.VMEM((3, m_per, n_per), jnp.int8),
            pltpu.VMEM((3, 2, 8, n_per), jnp.float32),
            pltpu.VMEM((3, m_per, n_per), jnp.int8),
            pltpu.VMEM((3, 2, 8, n_per), jnp.float32),
            pltpu.SemaphoreType.DMA((21,)),
            pltpu.SemaphoreType.DMA((21,)),
        ],
        compiler_params=pltpu.CompilerParams(
            collective_id=0, vmem_limit_bytes=60 * 1024 * 1024
        ),
    )(x, wb, qw, s_tile)
